# Initial kernel scaffold; baseline (speedup 1.0000x reference)
import jax
import jax.numpy as jnp
from jax.experimental import pallas as pl


def kernel(X, W_enc, b_enc, D, latent_bias, pre_bias):
    raise NotImplementedError("write your pallas kernel here")


# speedup vs baseline: 19.6627x; 19.6627x over previous
import functools

import jax
import jax.numpy as jnp
from jax.experimental import pallas as pl
from jax.experimental.pallas import tpu as pltpu

_K = 64


def _f32_sort_key(x):
    u = jax.lax.bitcast_convert_type(x, jnp.uint32)
    return jnp.where((u >> 31) == jnp.uint32(1), ~u, u | jnp.uint32(0x80000000))


def _enc_body(x_ref, w_ref, b_ref, lb_ref, pb_ref, s_ref, *, k):
    x = x_ref[...] - pb_ref[...]
    s_pre = jnp.dot(x, w_ref[...], preferred_element_type=jnp.float32,
                    precision=jax.lax.Precision.DEFAULT)
    s_pre = s_pre + b_ref[...] + lb_ref[...]
    key = _f32_sort_key(s_pre)

    def body(i, t):
        t_try = t | jax.lax.shift_left(jnp.uint32(1),
                                       (jnp.uint32(31) - i.astype(jnp.uint32)))
        cnt = jnp.sum((key >= t_try).astype(jnp.int32), axis=1, keepdims=True)
        return jnp.where(cnt >= k, t_try, t)

    t0 = jnp.zeros((x.shape[0], 1), jnp.uint32)
    kth = jax.lax.fori_loop(0, 32, body, t0)
    s_ref[...] = jnp.where(key >= kth, jnp.maximum(s_pre, 0.0), 0.0)


def _dec_body(s_ref, d_ref, pb_ref, r_ref, inv_ref):
    @pl.when(pl.program_id(0) == 0)
    def _():
        d = d_ref[...]
        inv_ref[...] = jax.lax.rsqrt(jnp.sum(d * d, axis=0, keepdims=True))

    r = jnp.dot(s_ref[...], d_ref[...], preferred_element_type=jnp.float32,
                precision=jax.lax.Precision.DEFAULT)
    r_ref[...] = r * inv_ref[...] + pb_ref[...]


def kernel(X, W_enc, b_enc, D, latent_bias, pre_bias):
    T, M = X.shape
    L = W_enc.shape[1]
    BT = 128

    S = pl.pallas_call(
        functools.partial(_enc_body, k=_K),
        grid=(T // BT,),
        in_specs=[
            pl.BlockSpec((BT, M), lambda i: (i, 0)),
            pl.BlockSpec((M, L), lambda i: (0, 0)),
            pl.BlockSpec((1, L), lambda i: (0, 0)),
            pl.BlockSpec((1, L), lambda i: (0, 0)),
            pl.BlockSpec((1, M), lambda i: (0, 0)),
        ],
        out_specs=pl.BlockSpec((BT, L), lambda i: (i, 0)),
        out_shape=jax.ShapeDtypeStruct((T, L), jnp.float32),
    )(X, W_enc, b_enc.reshape(1, L), latent_bias.reshape(1, L),
      pre_bias.reshape(1, M))

    X_recon = pl.pallas_call(
        _dec_body,
        grid=(T // BT,),
        in_specs=[
            pl.BlockSpec((BT, L), lambda i: (i, 0)),
            pl.BlockSpec((L, M), lambda i: (0, 0)),
            pl.BlockSpec((1, M), lambda i: (0, 0)),
        ],
        out_specs=pl.BlockSpec((BT, M), lambda i: (i, 0)),
        out_shape=jax.ShapeDtypeStruct((T, M), jnp.float32),
        scratch_shapes=[pltpu.VMEM((1, M), jnp.float32)],
    )(S, D, pre_bias.reshape(1, M))

    return (S, X_recon)
